# baseline (speedup 1.0000x reference)
import functools
import jax
import jax.numpy as jnp
from jax import lax
from jax.experimental import pallas as pl
from jax.experimental.pallas import tpu as pltpu
from jax.experimental.pallas import tpu_sc as plsc

_N = 4096
_K1 = 17
_K = 16
_BR = 256
_GRID = _N // _BR

_NC = 2
_NS = 16
_NW = _NC * _NS
_RPW = _N // _NW
_F = 3 * _N
_FS = _F // _NS


def _tc_body(p4_ref, post_ref, dist_ref, idx_ref):
    i = pl.program_id(0)
    base = i * _BR

    acc = jnp.zeros((_BR, _N), jnp.float32)
    for d in range(3):
        row = post_ref[d:d + 1, :]
        col = p4_ref[pl.ds(base, _BR), d:d + 1]
        diff = row - col
        acc = acc + diff * diff
    cur = jnp.sqrt(acc + 1e-8)

    iota = lax.broadcasted_iota(jnp.int32, (_BR, _N), 1)
    iota_k = lax.broadcasted_iota(jnp.int32, (_BR, _K1 + 15), 1)

    def step(m, carry):
        cur, mins, idxs = carry
        mn = jnp.min(cur, axis=1, keepdims=True)
        idx = jnp.argmin(cur, axis=1).astype(jnp.int32)[:, None]
        hot = iota_k == m
        mins = mins + jnp.where(hot, mn, 0.0)
        idxs = idxs + jnp.where(hot, idx, 0)
        cur = jnp.where(iota == idx, jnp.inf, cur)
        return cur, mins, idxs

    carry = (cur,
             jnp.zeros((_BR, _K1 + 15), jnp.float32),
             jnp.zeros((_BR, _K1 + 15), jnp.int32))
    _, mins, idxs = lax.fori_loop(0, _K1, step, carry)
    dist_ref[...] = mins
    idx_ref[...] = idxs


def _tc_topk(p4, post):
    return pl.pallas_call(
        _tc_body,
        grid=(_GRID,),
        in_specs=[
            pl.BlockSpec((_N, 4), lambda i: (0, 0)),
            pl.BlockSpec((3, _N), lambda i: (0, 0)),
        ],
        out_specs=[
            pl.BlockSpec((_BR, _K1 + 15), lambda i: (i, 0)),
            pl.BlockSpec((_BR, _K1 + 15), lambda i: (i, 0)),
        ],
        out_shape=[
            jax.ShapeDtypeStruct((_N, _K1 + 15), jnp.float32),
            jax.ShapeDtypeStruct((_N, _K1 + 15), jnp.int32),
        ],
    )(p4, post)


def _sc_grad_kernel(idx_hbm, dist_hbm, pos_hbm, out_hbm,
                    idx_v, dist_v, pos_v, g_v, tbuf_v, rbuf_v, shared):
    c = lax.axis_index("c")
    s = lax.axis_index("s")
    wid = s * _NC + c
    base_row = wid * _RPW

    pltpu.sync_copy(idx_hbm.at[pl.ds(base_row * _K, _RPW * _K)], idx_v)
    pltpu.sync_copy(dist_hbm.at[pl.ds(base_row * _K, _RPW * _K)], dist_v)
    pltpu.sync_copy(pos_hbm, pos_v)

    zero16 = jnp.zeros((16,), jnp.float32)

    def zloop(q, _):
        g_v[pl.ds(q * 16, 16)] = zero16
        return 0

    lax.fori_loop(0, _F // 16, zloop, 0)

    lane = lax.broadcasted_iota(jnp.int32, (16,), 0)

    def row_loop(r, carry):
        cx, cy, cz, cb = carry
        i_loc = cb * 16 + r
        off = i_loc * _K
        j = idx_v[pl.ds(off, 16)]
        dd = dist_v[pl.ds(off, 16)]
        wv = 1.0 / dd
        rv = jnp.zeros((16,), jnp.int32) + (base_row + i_loc)
        pix = plsc.load_gather(pos_v, [rv])
        piy = plsc.load_gather(pos_v, [rv + _N])
        piz = plsc.load_gather(pos_v, [rv + 2 * _N])
        gx = plsc.load_gather(pos_v, [j])
        gy = plsc.load_gather(pos_v, [j + _N])
        gz = plsc.load_gather(pos_v, [j + 2 * _N])
        vx = (gx - pix) * wv
        vy = (gy - piy) * wv
        vz = (gz - piz) * wv
        plsc.addupdate_scatter(g_v, [j], vx)
        plsc.addupdate_scatter(g_v, [j + _N], vy)
        plsc.addupdate_scatter(g_v, [j + 2 * _N], vz)
        hot = lane == r
        cx = cx + jnp.where(hot, -jnp.sum(vx), 0.0)
        cy = cy + jnp.where(hot, -jnp.sum(vy), 0.0)
        cz = cz + jnp.where(hot, -jnp.sum(vz), 0.0)
        return cx, cy, cz, cb

    def chunk_loop(cb, _):
        cx, cy, cz, _ = lax.fori_loop(
            0, 16, row_loop, (zero16, zero16, zero16, cb))
        o = base_row + cb * 16
        g_v[pl.ds(o, 16)] = g_v[pl.ds(o, 16)] + cx
        g_v[pl.ds(o + _N, 16)] = g_v[pl.ds(o + _N, 16)] + cy
        g_v[pl.ds(o + 2 * _N, 16)] = g_v[pl.ds(o + 2 * _N, 16)] + cz
        return 0

    lax.fori_loop(0, _RPW // 16, chunk_loop, 0)

    pltpu.sync_copy(g_v, shared.at[s])
    plsc.subcore_barrier()

    def zr(q, _):
        rbuf_v[pl.ds(q * 16, 16)] = zero16
        return 0

    lax.fori_loop(0, _FS // 16, zr, 0)

    def red_tile(t, _):
        pltpu.sync_copy(shared.at[t, pl.ds(s * _FS, _FS)], tbuf_v)

        def addq(q, _):
            rbuf_v[pl.ds(q * 16, 16)] = (
                rbuf_v[pl.ds(q * 16, 16)] + tbuf_v[pl.ds(q * 16, 16)])
            return 0

        lax.fori_loop(0, _FS // 16, addq, 0)
        return 0

    lax.fori_loop(0, _NS, red_tile, 0)
    pltpu.sync_copy(rbuf_v, out_hbm.at[c, pl.ds(s * _FS, _FS)])


@functools.partial(jax.jit, static_argnames=())
def _sc_grad(idx_flat, dist_flat, pos_flat):
    mesh = plsc.VectorSubcoreMesh(core_axis_name="c", subcore_axis_name="s")
    k = pl.kernel(
        _sc_grad_kernel,
        mesh=mesh,
        compiler_params=pltpu.CompilerParams(needs_layout_passes=False),
        out_type=jax.ShapeDtypeStruct((_NC, _F), jnp.float32),
        scratch_types=[
            pltpu.VMEM((_RPW * _K,), jnp.int32),
            pltpu.VMEM((_RPW * _K,), jnp.float32),
            pltpu.VMEM((_F,), jnp.float32),
            pltpu.VMEM((_F,), jnp.float32),
            pltpu.VMEM((_FS,), jnp.float32),
            pltpu.VMEM((_FS,), jnp.float32),
            pltpu.VMEM_SHARED((_NS, _F), jnp.float32),
        ],
    )
    return k(idx_flat, dist_flat, pos_flat)


def kernel(positions, batch):
    pos = positions.astype(jnp.float32)
    p4 = jnp.concatenate([pos, jnp.ones((_N, 1), jnp.float32)], axis=1)
    post = pos.T

    mins, idxs = _tc_topk(p4, post)
    dist16 = mins[:, 1:_K1]
    idx16 = idxs[:, 1:_K1]

    partials = _sc_grad(idx16.reshape(-1),
                        dist16.reshape(-1),
                        post.reshape(-1))
    g = (partials[0] + partials[1]).reshape(3, _N)
    return (dist16.reshape(1, -1), (g.T,))


# speedup vs baseline: 1.1531x; 1.1531x over previous
import functools
import jax
import jax.numpy as jnp
from jax import lax
from jax.experimental import pallas as pl
from jax.experimental.pallas import tpu as pltpu
from jax.experimental.pallas import tpu_sc as plsc

_N = 4096
_K1 = 17
_K = 16
_BR = 512
_GRID = _N // _BR

_NC = 2
_NS = 16
_NW = _NC * _NS
_RPW = _N // _NW
_F = 3 * _N
_FS = _F // _NS


def _tc_body(p4_ref, post_ref, dist_ref, idx_ref):
    i = pl.program_id(0)
    base = i * _BR

    acc = jnp.zeros((_BR, _N), jnp.float32)
    for d in range(3):
        row = post_ref[d:d + 1, :]
        col = p4_ref[pl.ds(base, _BR), d:d + 1]
        diff = row - col
        acc = acc + diff * diff
    cur = jnp.sqrt(acc + 1e-8)

    iota = lax.broadcasted_iota(jnp.int32, (_BR, _N), 1)
    iota_k = lax.broadcasted_iota(jnp.int32, (_BR, _K1 + 15), 1)

    def step(m, carry):
        cur, mins, idxs = carry
        mn = jnp.min(cur, axis=1, keepdims=True)
        idx = jnp.min(jnp.where(cur == mn, iota, _N), axis=1, keepdims=True)
        hot = iota_k == m
        mins = mins + jnp.where(hot, mn, 0.0)
        idxs = idxs + jnp.where(hot, idx, 0)
        cur = jnp.where(iota == idx, jnp.inf, cur)
        return cur, mins, idxs

    carry = (cur,
             jnp.zeros((_BR, _K1 + 15), jnp.float32),
             jnp.zeros((_BR, _K1 + 15), jnp.int32))
    _, mins, idxs = lax.fori_loop(0, _K1, step, carry)
    dist_ref[...] = mins
    idx_ref[...] = idxs


def _tc_topk(p4, post):
    return pl.pallas_call(
        _tc_body,
        grid=(_GRID,),
        in_specs=[
            pl.BlockSpec((_N, 4), lambda i: (0, 0)),
            pl.BlockSpec((3, _N), lambda i: (0, 0)),
        ],
        out_specs=[
            pl.BlockSpec((_BR, _K1 + 15), lambda i: (i, 0)),
            pl.BlockSpec((_BR, _K1 + 15), lambda i: (i, 0)),
        ],
        out_shape=[
            jax.ShapeDtypeStruct((_N, _K1 + 15), jnp.float32),
            jax.ShapeDtypeStruct((_N, _K1 + 15), jnp.int32),
        ],
    )(p4, post)


def _sc_grad_kernel(idx_hbm, dist_hbm, pos_hbm, out_hbm,
                    idx_v, dist_v, pos_v, g_v, tbuf_v, rbuf_v, shared):
    c = lax.axis_index("c")
    s = lax.axis_index("s")
    wid = s * _NC + c
    base_row = wid * _RPW

    pltpu.sync_copy(idx_hbm.at[pl.ds(base_row * _K, _RPW * _K)], idx_v)
    pltpu.sync_copy(dist_hbm.at[pl.ds(base_row * _K, _RPW * _K)], dist_v)
    pltpu.sync_copy(pos_hbm, pos_v)

    zero16 = jnp.zeros((16,), jnp.float32)

    def zloop(q, _):
        g_v[pl.ds(q * 16, 16)] = zero16
        return 0

    lax.fori_loop(0, _F // 16, zloop, 0)

    lane = lax.broadcasted_iota(jnp.int32, (16,), 0)

    def row_loop(r, carry):
        cx, cy, cz, cb = carry
        i_loc = cb * 16 + r
        off = i_loc * _K
        j = idx_v[pl.ds(off, 16)]
        dd = dist_v[pl.ds(off, 16)]
        wv = 1.0 / dd
        rv = jnp.zeros((16,), jnp.int32) + (base_row + i_loc)
        pix = plsc.load_gather(pos_v, [rv])
        piy = plsc.load_gather(pos_v, [rv + _N])
        piz = plsc.load_gather(pos_v, [rv + 2 * _N])
        gx = plsc.load_gather(pos_v, [j])
        gy = plsc.load_gather(pos_v, [j + _N])
        gz = plsc.load_gather(pos_v, [j + 2 * _N])
        vx = (gx - pix) * wv
        vy = (gy - piy) * wv
        vz = (gz - piz) * wv
        plsc.addupdate_scatter(g_v, [j], vx)
        plsc.addupdate_scatter(g_v, [j + _N], vy)
        plsc.addupdate_scatter(g_v, [j + 2 * _N], vz)
        hot = lane == r
        cx = cx + jnp.where(hot, -jnp.sum(vx), 0.0)
        cy = cy + jnp.where(hot, -jnp.sum(vy), 0.0)
        cz = cz + jnp.where(hot, -jnp.sum(vz), 0.0)
        return cx, cy, cz, cb

    def chunk_loop(cb, _):
        cx, cy, cz, _ = lax.fori_loop(
            0, 16, row_loop, (zero16, zero16, zero16, cb))
        o = base_row + cb * 16
        g_v[pl.ds(o, 16)] = g_v[pl.ds(o, 16)] + cx
        g_v[pl.ds(o + _N, 16)] = g_v[pl.ds(o + _N, 16)] + cy
        g_v[pl.ds(o + 2 * _N, 16)] = g_v[pl.ds(o + 2 * _N, 16)] + cz
        return 0

    lax.fori_loop(0, _RPW // 16, chunk_loop, 0)

    pltpu.sync_copy(g_v, shared.at[s])
    plsc.subcore_barrier()

    def zr(q, _):
        rbuf_v[pl.ds(q * 16, 16)] = zero16
        return 0

    lax.fori_loop(0, _FS // 16, zr, 0)

    def red_tile(t, _):
        pltpu.sync_copy(shared.at[t, pl.ds(s * _FS, _FS)], tbuf_v)

        def addq(q, _):
            rbuf_v[pl.ds(q * 16, 16)] = (
                rbuf_v[pl.ds(q * 16, 16)] + tbuf_v[pl.ds(q * 16, 16)])
            return 0

        lax.fori_loop(0, _FS // 16, addq, 0)
        return 0

    lax.fori_loop(0, _NS, red_tile, 0)
    pltpu.sync_copy(rbuf_v, out_hbm.at[c, pl.ds(s * _FS, _FS)])


@functools.partial(jax.jit, static_argnames=())
def _sc_grad(idx_flat, dist_flat, pos_flat):
    mesh = plsc.VectorSubcoreMesh(core_axis_name="c", subcore_axis_name="s")
    k = pl.kernel(
        _sc_grad_kernel,
        mesh=mesh,
        compiler_params=pltpu.CompilerParams(needs_layout_passes=False),
        out_type=jax.ShapeDtypeStruct((_NC, _F), jnp.float32),
        scratch_types=[
            pltpu.VMEM((_RPW * _K,), jnp.int32),
            pltpu.VMEM((_RPW * _K,), jnp.float32),
            pltpu.VMEM((_F,), jnp.float32),
            pltpu.VMEM((_F,), jnp.float32),
            pltpu.VMEM((_FS,), jnp.float32),
            pltpu.VMEM((_FS,), jnp.float32),
            pltpu.VMEM_SHARED((_NS, _F), jnp.float32),
        ],
    )
    return k(idx_flat, dist_flat, pos_flat)


def kernel(positions, batch):
    pos = positions.astype(jnp.float32)
    p4 = jnp.concatenate([pos, jnp.ones((_N, 1), jnp.float32)], axis=1)
    post = pos.T

    mins, idxs = _tc_topk(p4, post)
    dist16 = mins[:, 1:_K1]
    idx16 = idxs[:, 1:_K1]

    partials = _sc_grad(idx16.reshape(-1),
                        dist16.reshape(-1),
                        post.reshape(-1))
    g = (partials[0] + partials[1]).reshape(3, _N)
    return (dist16.reshape(1, -1), (g.T,))
